# Initial kernel scaffold; baseline (speedup 1.0000x reference)
import functools

import jax
import jax.numpy as jnp
from jax import lax
from jax.experimental import pallas as pl
from jax.experimental.pallas import tpu as pltpu
from jax.experimental.pallas import tpu_sc as plsc

N = 10000
E = 160000
D = 128
ORDER = 3
LANES = 16

NC = 2
NS = 16
NW = NC * NS
EPT = E // NW
CHUNK = 40
NCHUNK = EPT // CHUNK
RPT = N // NS
ZROWS = 125
BN = 1000
GRID = N // BN

assert E == NW * EPT and EPT == NCHUNK * CHUNK and CHUNK % 8 == 0
assert N == NS * RPT and RPT == 5 * ZROWS

_mesh = plsc.VectorSubcoreMesh(core_axis_name="c", subcore_axis_name="s")


@functools.partial(
    pl.kernel,
    out_type=jax.ShapeDtypeStruct((NC, ORDER, N, LANES), jnp.float32),
    mesh=_mesh,
    scratch_types=[
        pltpu.VMEM_SHARED((N, LANES), jnp.float32),
        pltpu.VMEM_SHARED((N, LANES), jnp.float32),
        pltpu.VMEM_SHARED((N, LANES), jnp.float32),
        pltpu.VMEM((CHUNK, LANES), jnp.float32),
        pltpu.VMEM((ZROWS, LANES), jnp.float32),
        pltpu.VMEM((CHUNK,), jnp.int32),
        pltpu.VMEM((ZROWS, LANES), jnp.float32),
    ],
)
def _deg_kernel(dst0, dst1, dst2, ones_hbm, zeros_hbm, out,
                acc0, acc1, acc2, ones_b, zb, didx, bb):
    c = lax.axis_index("c")
    s = lax.axis_index("s")
    w = c * NS + s
    accs = (acc0, acc1, acc2)
    dsts = (dst0, dst1, dst2)

    pltpu.sync_copy(ones_hbm, ones_b)
    pltpu.sync_copy(zeros_hbm, zb)
    for acc in accs:
        for j in range(RPT // ZROWS):
            pltpu.sync_copy(zb, acc.at[pl.ds(s * RPT + j * ZROWS, ZROWS)])
    plsc.subcore_barrier()

    for o in range(ORDER):
        @pl.loop(0, NCHUNK)
        def _(k, o=o):
            base = w * EPT + k * CHUNK
            pltpu.sync_copy(dsts[o].at[pl.ds(base, CHUNK)], didx)
            pltpu.sync_copy(ones_b, accs[o].at[didx], add=True)
    plsc.subcore_barrier()

    for o in range(ORDER):
        for j in range(RPT // ZROWS):
            r0 = s * RPT + j * ZROWS
            pltpu.sync_copy(accs[o].at[pl.ds(r0, ZROWS)], bb)
            pltpu.sync_copy(bb, out.at[c, o, pl.ds(r0, ZROWS)])


@functools.partial(
    pl.kernel,
    out_type=jax.ShapeDtypeStruct((NC, ORDER, N, D), jnp.float32),
    mesh=_mesh,
    scratch_types=[
        pltpu.VMEM_SHARED((N, D), jnp.float32),
        pltpu.VMEM((ZROWS, D), jnp.float32),
        pltpu.VMEM((CHUNK,), jnp.int32),
        pltpu.VMEM((CHUNK,), jnp.int32),
        pltpu.VMEM((CHUNK, D), jnp.float32),
        pltpu.VMEM((ZROWS, D), jnp.float32),
        pltpu.SemaphoreType.DMA,
    ],
)
def _spmm_kernel(h0, h1, h2, src0, dst0, src1, dst1, src2, dst2, zeros_hbm,
                 out, acc, zb, sidx, didx, rows, bb, sem):
    c = lax.axis_index("c")
    s = lax.axis_index("s")
    w = c * NS + s
    hs = (h0, h1, h2)
    srcs = (src0, src1, src2)
    dsts = (dst0, dst1, dst2)

    pltpu.sync_copy(zeros_hbm, zb)
    for o in range(ORDER):
        for j in range(RPT // ZROWS):
            pltpu.sync_copy(zb, acc.at[pl.ds(s * RPT + j * ZROWS, ZROWS)])
        plsc.subcore_barrier()

        @pl.loop(0, NCHUNK)
        def _(k, o=o):
            base = w * EPT + k * CHUNK
            pltpu.sync_copy(srcs[o].at[pl.ds(base, CHUNK)], sidx)
            pltpu.sync_copy(dsts[o].at[pl.ds(base, CHUNK)], didx)
            pltpu.async_copy(hs[o].at[sidx], rows, sem).wait()
            pltpu.sync_copy(rows, acc.at[didx], add=True)
        plsc.subcore_barrier()

        for j in range(RPT // ZROWS):
            r0 = s * RPT + j * ZROWS
            pltpu.sync_copy(acc.at[pl.ds(r0, ZROWS)], bb)
            pltpu.sync_copy(bb, out.at[c, o, pl.ds(r0, ZROWS)])
        plsc.subcore_barrier()


def _elu(x):
    return jnp.where(x > 0, x, jnp.exp(x) - 1.0)


def _tc1_body(x_ref, degp_ref, wcat_ref, h0_ref, h1_ref, h2_ref, rt_ref):
    h = jnp.dot(x_ref[...], wcat_ref[...], preferred_element_type=jnp.float32)
    hs = (h0_ref, h1_ref, h2_ref)
    for o in range(ORDER):
        deg = degp_ref[0, o, :, 0:1] + degp_ref[1, o, :, 0:1]
        r = lax.rsqrt(jnp.maximum(deg, 1.0))
        hs[o][...] = h[:, o * D:(o + 1) * D] * r
        rt_ref[:, o:o + 1] = r


def _tc_mid(p_ref, rt_ref, bcat_ref, a0_ref, c0_ref, a1_ref, c1_ref):
    ss = []
    for o in range(ORDER):
        r = rt_ref[:, o:o + 1]
        t = (p_ref[0, o] + p_ref[1, o]) * r + bcat_ref[:, o * D:(o + 1) * D]
        ss.append(_elu(t))
    cat = jnp.concatenate(ss, axis=1)
    u = _elu(jnp.dot(cat, a0_ref[...], preferred_element_type=jnp.float32)
             + c0_ref[...])
    return jnp.dot(u, a1_ref[...], preferred_element_type=jnp.float32) + c1_ref[...]


def _tc2_body(p_ref, rt_ref, bcat_ref, a0_ref, c0_ref, a1_ref, c1_ref,
              wcat_ref, h0_ref, h1_ref, h2_ref):
    st = _tc_mid(p_ref, rt_ref, bcat_ref, a0_ref, c0_ref, a1_ref, c1_ref)
    h = jnp.dot(st, wcat_ref[...], preferred_element_type=jnp.float32)
    hs = (h0_ref, h1_ref, h2_ref)
    for o in range(ORDER):
        hs[o][...] = h[:, o * D:(o + 1) * D] * rt_ref[:, o:o + 1]


def _tc3_body(p_ref, rt_ref, bcat_ref, a0_ref, c0_ref, a1_ref, c1_ref,
              out_ref):
    st = _tc_mid(p_ref, rt_ref, bcat_ref, a0_ref, c0_ref, a1_ref, c1_ref)
    m = jnp.max(st, axis=1, keepdims=True)
    e = st - m
    lse = jnp.log(jnp.sum(jnp.exp(e), axis=1, keepdims=True))
    out_ref[...] = e - lse


def _row_spec(cols):
    return pl.BlockSpec((BN, cols), lambda i: (i, 0))


def _full_spec(shape):
    nd = len(shape)
    return pl.BlockSpec(shape, lambda i, nd=nd: (0,) * nd)


_nd_f32 = jax.ShapeDtypeStruct((N, D), jnp.float32)

_tc1 = pl.pallas_call(
    _tc1_body,
    grid=(GRID,),
    in_specs=[
        _row_spec(D),
        pl.BlockSpec((NC, ORDER, BN, LANES), lambda i: (0, 0, i, 0)),
        _full_spec((D, ORDER * D)),
    ],
    out_specs=[_row_spec(D)] * 3 + [_row_spec(ORDER)],
    out_shape=[_nd_f32] * 3 + [jax.ShapeDtypeStruct((N, ORDER), jnp.float32)],
)

_mid_specs = [
    pl.BlockSpec((NC, ORDER, BN, D), lambda i: (0, 0, i, 0)),
    _row_spec(ORDER),
    _full_spec((1, ORDER * D)),
    _full_spec((ORDER * D, D)),
    _full_spec((1, D)),
    _full_spec((D, D)),
    _full_spec((1, D)),
]

_tc2 = pl.pallas_call(
    _tc2_body,
    grid=(GRID,),
    in_specs=_mid_specs + [_full_spec((D, ORDER * D))],
    out_specs=[_row_spec(D)] * 3,
    out_shape=[_nd_f32] * 3,
)

_tc3 = pl.pallas_call(
    _tc3_body,
    grid=(GRID,),
    in_specs=_mid_specs,
    out_specs=_row_spec(D),
    out_shape=_nd_f32,
)


def kernel(node_feature, adj0, adj1, adj2,
           W00, b00, W01, b01, W02, b02, A0_0, c0_0, A1_0, c1_0,
           W10, b10, W11, b11, W12, b12, A0_1, c0_1, A1_1, c1_1):
    srcs = [adj0[0], adj1[0], adj2[0]]
    dsts = [adj0[1], adj1[1], adj2[1]]
    ones16 = jnp.ones((CHUNK, LANES), jnp.float32)
    zeros16 = jnp.zeros((ZROWS, LANES), jnp.float32)
    zerosd = jnp.zeros((ZROWS, D), jnp.float32)

    degp = _deg_kernel(dsts[0], dsts[1], dsts[2], ones16, zeros16)

    wcat0 = jnp.concatenate([W00, W01, W02], axis=1)
    wcat1 = jnp.concatenate([W10, W11, W12], axis=1)
    bcat0 = jnp.concatenate([b00, b01, b02]).reshape(1, ORDER * D)
    bcat1 = jnp.concatenate([b10, b11, b12]).reshape(1, ORDER * D)

    h0, h1, h2, rt = _tc1(node_feature, degp, wcat0)
    p = _spmm_kernel(h0, h1, h2, srcs[0], dsts[0], srcs[1], dsts[1],
                     srcs[2], dsts[2], zerosd)
    h0, h1, h2 = _tc2(p, rt, bcat0, A0_0, c0_0.reshape(1, D), A1_0,
                      c1_0.reshape(1, D), wcat1)
    p = _spmm_kernel(h0, h1, h2, srcs[0], dsts[0], srcs[1], dsts[1],
                     srcs[2], dsts[2], zerosd)
    return _tc3(p, rt, bcat1, A0_1, c0_1.reshape(1, D), A1_1,
                c1_1.reshape(1, D))


# speedup vs baseline: 5.7182x; 5.7182x over previous
import functools

import jax
import jax.numpy as jnp
from jax import lax
from jax.experimental import pallas as pl
from jax.experimental.pallas import tpu as pltpu
from jax.experimental.pallas import tpu_sc as plsc

N = 10000
E = 160000
D = 128
ORDER = 3
LANES = 16

NC = 2
NS = 16
NW = NC * NS
EPT = E // NW
CHUNK = 40
NCHUNK = EPT // CHUNK
N_PAD = 10240
RPT = N_PAD // NS
ZROWS = 128
BN = 1000
GRID = N // BN

assert E == NW * EPT and EPT == NCHUNK * CHUNK and CHUNK % 8 == 0
assert N_PAD == NS * RPT and RPT == 5 * ZROWS and RPT % 8 == 0

def _mesh():
    return plsc.VectorSubcoreMesh(core_axis_name="c", subcore_axis_name="s",
                                  num_cores=NC, num_subcores=NS)


def _deg_body(dst0, dst1, dst2, ones_hbm, zeros_hbm, out,
              acc, ones_b, zb, didx, bb):
    c = lax.axis_index("c")
    s = lax.axis_index("s")
    w = c * NS + s
    dsts = (dst0, dst1, dst2)

    pltpu.sync_copy(ones_hbm, ones_b)
    pltpu.sync_copy(zeros_hbm, zb)
    for o in range(ORDER):
        for j in range(RPT // ZROWS):
            pltpu.sync_copy(zb, acc.at[pl.ds(s * RPT + j * ZROWS, ZROWS)])
        plsc.subcore_barrier()

        @pl.loop(0, NCHUNK)
        def _(k, o=o):
            base = w * EPT + k * CHUNK
            pltpu.sync_copy(dsts[o].at[pl.ds(base, CHUNK)], didx)
            pltpu.sync_copy(ones_b, acc.at[didx], add=True)
        plsc.subcore_barrier()

        for j in range(RPT // ZROWS):
            r0 = s * RPT + j * ZROWS
            pltpu.sync_copy(acc.at[pl.ds(r0, ZROWS)], bb)
            pltpu.sync_copy(bb, out.at[c, o, pl.ds(r0, ZROWS)])
        plsc.subcore_barrier()


def _spmm_body(h0, h1, h2, src0, dst0, src1, dst1, src2, dst2, zeros_hbm,
               out, acc, zb, sidx, didx, rows, bb, sem):
    c = lax.axis_index("c")
    s = lax.axis_index("s")
    w = c * NS + s
    hs = (h0, h1, h2)
    srcs = (src0, src1, src2)
    dsts = (dst0, dst1, dst2)

    pltpu.sync_copy(zeros_hbm, zb)
    for o in range(ORDER):
        for j in range(RPT // ZROWS):
            pltpu.sync_copy(zb, acc.at[pl.ds(s * RPT + j * ZROWS, ZROWS)])
        plsc.subcore_barrier()

        @pl.loop(0, NCHUNK)
        def _(k, o=o):
            base = w * EPT + k * CHUNK
            pltpu.sync_copy(srcs[o].at[pl.ds(base, CHUNK)], sidx)
            pltpu.sync_copy(dsts[o].at[pl.ds(base, CHUNK)], didx)
            pltpu.async_copy(hs[o].at[sidx], rows, sem).wait()
            pltpu.sync_copy(rows, acc.at[didx], add=True)
        plsc.subcore_barrier()

        for j in range(RPT // ZROWS):
            r0 = s * RPT + j * ZROWS
            pltpu.sync_copy(acc.at[pl.ds(r0, ZROWS)], bb)
            pltpu.sync_copy(bb, out.at[c, o, pl.ds(r0, ZROWS)])
        plsc.subcore_barrier()


@functools.cache
def _deg_kernel():
    return pl.kernel(
        _deg_body,
        out_type=jax.ShapeDtypeStruct((NC, ORDER, N_PAD, D), jnp.float32),
        mesh=_mesh(),
        scratch_types=[
            pltpu.VMEM_SHARED((N_PAD, D), jnp.float32),
            pltpu.VMEM((CHUNK, D), jnp.float32),
            pltpu.VMEM((ZROWS, D), jnp.float32),
            pltpu.VMEM((CHUNK,), jnp.int32),
            pltpu.VMEM((ZROWS, D), jnp.float32),
        ],
    )


@functools.cache
def _spmm_kernel():
    return pl.kernel(
        _spmm_body,
        out_type=jax.ShapeDtypeStruct((NC, ORDER, N_PAD, D), jnp.float32),
        mesh=_mesh(),
        scratch_types=[
            pltpu.VMEM_SHARED((N_PAD, D), jnp.float32),
            pltpu.VMEM((ZROWS, D), jnp.float32),
            pltpu.VMEM((CHUNK,), jnp.int32),
            pltpu.VMEM((CHUNK,), jnp.int32),
            pltpu.VMEM((CHUNK, D), jnp.float32),
            pltpu.VMEM((ZROWS, D), jnp.float32),
            pltpu.SemaphoreType.DMA,
        ],
    )


def _elu(x):
    return jnp.where(x > 0, x, jnp.exp(x) - 1.0)


def _tc1_body(x_ref, degp_ref, wcat_ref, h0_ref, h1_ref, h2_ref, rt_ref):
    h = jnp.dot(x_ref[...], wcat_ref[...], preferred_element_type=jnp.float32)
    hs = (h0_ref, h1_ref, h2_ref)
    for o in range(ORDER):
        deg = degp_ref[0, o, :, 0:1] + degp_ref[1, o, :, 0:1]
        r = lax.rsqrt(jnp.maximum(deg, 1.0))
        hs[o][...] = h[:, o * D:(o + 1) * D] * r
        rt_ref[:, o:o + 1] = r


def _tc_mid(p_ref, rt_ref, bcat_ref, a0_ref, c0_ref, a1_ref, c1_ref):
    ss = []
    for o in range(ORDER):
        r = rt_ref[:, o:o + 1]
        t = (p_ref[0, o] + p_ref[1, o]) * r + bcat_ref[:, o * D:(o + 1) * D]
        ss.append(_elu(t))
    cat = jnp.concatenate(ss, axis=1)
    u = _elu(jnp.dot(cat, a0_ref[...], preferred_element_type=jnp.float32)
             + c0_ref[...])
    return jnp.dot(u, a1_ref[...], preferred_element_type=jnp.float32) + c1_ref[...]


def _tc2_body(p_ref, rt_ref, bcat_ref, a0_ref, c0_ref, a1_ref, c1_ref,
              wcat_ref, h0_ref, h1_ref, h2_ref):
    st = _tc_mid(p_ref, rt_ref, bcat_ref, a0_ref, c0_ref, a1_ref, c1_ref)
    h = jnp.dot(st, wcat_ref[...], preferred_element_type=jnp.float32)
    hs = (h0_ref, h1_ref, h2_ref)
    for o in range(ORDER):
        hs[o][...] = h[:, o * D:(o + 1) * D] * rt_ref[:, o:o + 1]


def _tc3_body(p_ref, rt_ref, bcat_ref, a0_ref, c0_ref, a1_ref, c1_ref,
              out_ref):
    st = _tc_mid(p_ref, rt_ref, bcat_ref, a0_ref, c0_ref, a1_ref, c1_ref)
    m = jnp.max(st, axis=1, keepdims=True)
    e = st - m
    lse = jnp.log(jnp.sum(jnp.exp(e), axis=1, keepdims=True))
    out_ref[...] = e - lse


def _row_spec(cols):
    return pl.BlockSpec((BN, cols), lambda i: (i, 0))


def _full_spec(shape):
    nd = len(shape)
    return pl.BlockSpec(shape, lambda i, nd=nd: (0,) * nd)


_nd_f32 = jax.ShapeDtypeStruct((N, D), jnp.float32)

_tc1 = pl.pallas_call(
    _tc1_body,
    grid=(GRID,),
    in_specs=[
        _row_spec(D),
        pl.BlockSpec((NC, ORDER, BN, D), lambda i: (0, 0, i, 0)),
        _full_spec((D, ORDER * D)),
    ],
    out_specs=[_row_spec(D)] * 3 + [_row_spec(ORDER)],
    out_shape=[_nd_f32] * 3 + [jax.ShapeDtypeStruct((N, ORDER), jnp.float32)],
)

_mid_specs = [
    pl.BlockSpec((NC, ORDER, BN, D), lambda i: (0, 0, i, 0)),
    _row_spec(ORDER),
    _full_spec((1, ORDER * D)),
    _full_spec((ORDER * D, D)),
    _full_spec((1, D)),
    _full_spec((D, D)),
    _full_spec((1, D)),
]

_tc2 = pl.pallas_call(
    _tc2_body,
    grid=(GRID,),
    in_specs=_mid_specs + [_full_spec((D, ORDER * D))],
    out_specs=[_row_spec(D)] * 3,
    out_shape=[_nd_f32] * 3,
)

_tc3 = pl.pallas_call(
    _tc3_body,
    grid=(GRID,),
    in_specs=_mid_specs,
    out_specs=_row_spec(D),
    out_shape=_nd_f32,
)


def kernel(node_feature, adj0, adj1, adj2,
           W00, b00, W01, b01, W02, b02, A0_0, c0_0, A1_0, c1_0,
           W10, b10, W11, b11, W12, b12, A0_1, c0_1, A1_1, c1_1):
    srcs = [adj0[0], adj1[0], adj2[0]]
    dsts = [adj0[1], adj1[1], adj2[1]]
    onesd = jnp.ones((CHUNK, D), jnp.float32)
    zerosd = jnp.zeros((ZROWS, D), jnp.float32)

    degp = _deg_kernel()(dsts[0], dsts[1], dsts[2], onesd, zerosd)

    wcat0 = jnp.concatenate([W00, W01, W02], axis=1)
    wcat1 = jnp.concatenate([W10, W11, W12], axis=1)
    bcat0 = jnp.concatenate([b00, b01, b02]).reshape(1, ORDER * D)
    bcat1 = jnp.concatenate([b10, b11, b12]).reshape(1, ORDER * D)

    h0, h1, h2, rt = _tc1(node_feature, degp, wcat0)
    p = _spmm_kernel()(h0, h1, h2, srcs[0], dsts[0], srcs[1], dsts[1],
                     srcs[2], dsts[2], zerosd)
    h0, h1, h2 = _tc2(p, rt, bcat0, A0_0, c0_0.reshape(1, D), A1_0,
                      c1_0.reshape(1, D), wcat1)
    p = _spmm_kernel()(h0, h1, h2, srcs[0], dsts[0], srcs[1], dsts[1],
                     srcs[2], dsts[2], zerosd)
    return _tc3(p, rt, bcat1, A0_1, c0_1.reshape(1, D), A1_1,
                c1_1.reshape(1, D))


# speedup vs baseline: 15.8623x; 2.7740x over previous
import functools

import jax
import jax.numpy as jnp
from jax import lax
from jax.experimental import pallas as pl
from jax.experimental.pallas import tpu as pltpu
from jax.experimental.pallas import tpu_sc as plsc

N = 10000
E = 160000
D = 128
ORDER = 3
LANES = 16

NC = 2
NS = 16
NW = NC * NS
EPT = E // NW
CHUNK = 128
NCHK = E // CHUNK
NK = NCHK // NW
EXTRA = NCHK - NK * NW
N_PAD = 10240
RPT = N_PAD // NS
ZROWS = 64
BN = 1000
GRID = N // BN

assert E == NCHK * CHUNK and CHUNK % 8 == 0 and CHUNK <= 128
assert N_PAD == NS * RPT and RPT % ZROWS == 0 and RPT % 8 == 0

def _mesh():
    return plsc.VectorSubcoreMesh(core_axis_name="c", subcore_axis_name="s",
                                  num_cores=NC, num_subcores=NS)


def _deg_body(dst0, dst1, dst2, ones_hbm, zeros_hbm, out,
              acc, ones_b, zb, didx0, didx1, isem0, isem1, ssem0, ssem1):
    c = lax.axis_index("c")
    s = lax.axis_index("s")
    w = c * NS + s
    dsts = (dst0, dst1, dst2)
    didx = (didx0, didx1)
    isem = (isem0, isem1)
    ssem = (ssem0, ssem1)

    pltpu.sync_copy(ones_hbm, ones_b)
    pltpu.sync_copy(zeros_hbm, zb)
    for o in range(ORDER):
        dst_r = dsts[o]

        def base_of(k):
            return (k * NW + w) * CHUNK

        def issue_idx(k, b, dst_r=dst_r):
            pltpu.async_copy(dst_r.at[pl.ds(base_of(k), CHUNK)], didx[b],
                             isem[b])

        def wait_idx(k, b, dst_r=dst_r):
            pltpu.make_async_copy(dst_r.at[pl.ds(base_of(k), CHUNK)],
                                  didx[b], isem[b]).wait()

        for j in range(RPT // ZROWS):
            pltpu.sync_copy(zb, acc.at[pl.ds(s * RPT + j * ZROWS, ZROWS)])
        plsc.subcore_barrier()

        issue_idx(0, 0)

        @pl.loop(0, NK // 2)
        def _(j, issue_idx=issue_idx, wait_idx=wait_idx):
            for b in (0, 1):
                k = j * 2 + b

                @pl.when(k > 0)
                def _():
                    pltpu.make_async_copy(ones_b, acc.at[didx[1 - b]],
                                          ssem[1 - b]).wait()
                issue_idx(k + 1, 1 - b)
                wait_idx(k, b)
                pltpu.async_copy(ones_b, acc.at[didx[b]], ssem[b], add=True)

        if NK % 2 == 1:
            kl = NK - 1
            pltpu.make_async_copy(ones_b, acc.at[didx[1]], ssem[1]).wait()
            wait_idx(kl, 0)
            pltpu.async_copy(ones_b, acc.at[didx[0]], ssem[0], add=True)
        pltpu.make_async_copy(ones_b, acc.at[didx[0]], ssem[0]).wait()

        @pl.when(w < EXTRA)
        def _(dst_r=dst_r):
            base = (NK * NW + w) * CHUNK
            pltpu.sync_copy(dst_r.at[pl.ds(base, CHUNK)], didx[1])
            pltpu.sync_copy(ones_b, acc.at[didx[1]], add=True)
        plsc.subcore_barrier()

        for j in range(RPT // ZROWS):
            r0 = s * RPT + j * ZROWS
            pltpu.sync_copy(acc.at[pl.ds(r0, ZROWS)], zb)
            pltpu.sync_copy(zb, out.at[c, o, pl.ds(r0, ZROWS)])
        plsc.subcore_barrier()
        pltpu.sync_copy(zeros_hbm, zb)


def _spmm_body(h0, h1, h2, src0, dst0, src1, dst1, src2, dst2, zeros_hbm,
               out, acc, zb, sidx0, sidx1, didx0, didx1, rows0, rows1,
               isem0, isem1, gsem0, gsem1, ssem0, ssem1):
    c = lax.axis_index("c")
    s = lax.axis_index("s")
    w = c * NS + s
    hs = (h0, h1, h2)
    srcs = (src0, src1, src2)
    dsts = (dst0, dst1, dst2)
    sidx = (sidx0, sidx1)
    didx = (didx0, didx1)
    rows = (rows0, rows1)
    isem = (isem0, isem1)
    gsem = (gsem0, gsem1)
    ssem = (ssem0, ssem1)

    pltpu.sync_copy(zeros_hbm, zb)
    for o in range(ORDER):
        src_r, dst_r, h_r = srcs[o], dsts[o], hs[o]

        def base_of(k):
            return (k * NW + w) * CHUNK

        def issue_idx(k, b, src_r=src_r, dst_r=dst_r):
            pltpu.async_copy(src_r.at[pl.ds(base_of(k), CHUNK)], sidx[b],
                             isem[b])
            pltpu.async_copy(dst_r.at[pl.ds(base_of(k), CHUNK)], didx[b],
                             isem[b])

        def wait_idx(k, b, src_r=src_r, dst_r=dst_r):
            pltpu.make_async_copy(src_r.at[pl.ds(base_of(k), CHUNK)],
                                  sidx[b], isem[b]).wait()
            pltpu.make_async_copy(dst_r.at[pl.ds(base_of(k), CHUNK)],
                                  didx[b], isem[b]).wait()

        for j in range(RPT // ZROWS):
            pltpu.sync_copy(zb, acc.at[pl.ds(s * RPT + j * ZROWS, ZROWS)])
        plsc.subcore_barrier()

        issue_idx(0, 0)

        @pl.loop(0, NK // 2)
        def _(j, issue_idx=issue_idx, wait_idx=wait_idx, h_r=h_r):
            for b in (0, 1):
                k = j * 2 + b
                wait_idx(k, b)
                pltpu.async_copy(h_r.at[sidx[b]], rows[b], gsem[b])

                @pl.when(k > 0)
                def _():
                    pltpu.make_async_copy(rows[1 - b], acc.at[didx[1 - b]],
                                          ssem[1 - b]).wait()
                issue_idx(k + 1, 1 - b)
                pltpu.make_async_copy(h_r.at[sidx[b]], rows[b],
                                      gsem[b]).wait()
                pltpu.async_copy(rows[b], acc.at[didx[b]], ssem[b], add=True)

        if NK % 2 == 1:
            kl = NK - 1
            wait_idx(kl, 0)
            pltpu.async_copy(h_r.at[sidx[0]], rows[0], gsem[0])
            pltpu.make_async_copy(rows[1], acc.at[didx[1]], ssem[1]).wait()
            pltpu.make_async_copy(h_r.at[sidx[0]], rows[0], gsem[0]).wait()
            pltpu.async_copy(rows[0], acc.at[didx[0]], ssem[0], add=True)
        pltpu.make_async_copy(rows[0], acc.at[didx[0]], ssem[0]).wait()

        @pl.when(w < EXTRA)
        def _(src_r=src_r, dst_r=dst_r, h_r=h_r):
            base = (NK * NW + w) * CHUNK
            pltpu.sync_copy(src_r.at[pl.ds(base, CHUNK)], sidx[1])
            pltpu.sync_copy(dst_r.at[pl.ds(base, CHUNK)], didx[1])
            pltpu.async_copy(h_r.at[sidx[1]], rows[1], gsem[1]).wait()
            pltpu.sync_copy(rows[1], acc.at[didx[1]], add=True)
        plsc.subcore_barrier()

        for j in range(RPT // ZROWS):
            r0 = s * RPT + j * ZROWS
            pltpu.sync_copy(acc.at[pl.ds(r0, ZROWS)], zb)
            pltpu.sync_copy(zb, out.at[c, o, pl.ds(r0, ZROWS)])
        plsc.subcore_barrier()
        pltpu.sync_copy(zeros_hbm, zb)


@functools.cache
def _deg_kernel():
    return pl.kernel(
        _deg_body,
        out_type=jax.ShapeDtypeStruct((NC, ORDER, N_PAD, D), jnp.float32),
        mesh=_mesh(),
        scratch_types=[
            pltpu.VMEM_SHARED((N_PAD, D), jnp.float32),
            pltpu.VMEM((CHUNK, D), jnp.float32),
            pltpu.VMEM((ZROWS, D), jnp.float32),
            pltpu.VMEM((CHUNK,), jnp.int32),
            pltpu.VMEM((CHUNK,), jnp.int32),
            pltpu.SemaphoreType.DMA,
            pltpu.SemaphoreType.DMA,
            pltpu.SemaphoreType.DMA,
            pltpu.SemaphoreType.DMA,
        ],
    )


@functools.cache
def _spmm_kernel():
    return pl.kernel(
        _spmm_body,
        out_type=jax.ShapeDtypeStruct((NC, ORDER, N_PAD, D), jnp.float32),
        mesh=_mesh(),
        scratch_types=[
            pltpu.VMEM_SHARED((N_PAD, D), jnp.float32),
            pltpu.VMEM((ZROWS, D), jnp.float32),
            pltpu.VMEM((CHUNK,), jnp.int32),
            pltpu.VMEM((CHUNK,), jnp.int32),
            pltpu.VMEM((CHUNK,), jnp.int32),
            pltpu.VMEM((CHUNK,), jnp.int32),
            pltpu.VMEM((CHUNK, D), jnp.float32),
            pltpu.VMEM((CHUNK, D), jnp.float32),
            pltpu.SemaphoreType.DMA,
            pltpu.SemaphoreType.DMA,
            pltpu.SemaphoreType.DMA,
            pltpu.SemaphoreType.DMA,
            pltpu.SemaphoreType.DMA,
            pltpu.SemaphoreType.DMA,
        ],
    )


def _elu(x):
    return jnp.where(x > 0, x, jnp.exp(x) - 1.0)


def _tc1_body(x_ref, degp_ref, wcat_ref, h0_ref, h1_ref, h2_ref, rt_ref):
    h = jnp.dot(x_ref[...], wcat_ref[...], preferred_element_type=jnp.float32)
    hs = (h0_ref, h1_ref, h2_ref)
    for o in range(ORDER):
        deg = degp_ref[0, o, :, 0:1] + degp_ref[1, o, :, 0:1]
        r = lax.rsqrt(jnp.maximum(deg, 1.0))
        hs[o][...] = h[:, o * D:(o + 1) * D] * r
        rt_ref[:, o:o + 1] = r


def _tc_mid(p_ref, rt_ref, bcat_ref, a0_ref, c0_ref, a1_ref, c1_ref):
    ss = []
    for o in range(ORDER):
        r = rt_ref[:, o:o + 1]
        t = (p_ref[0, o] + p_ref[1, o]) * r + bcat_ref[:, o * D:(o + 1) * D]
        ss.append(_elu(t))
    cat = jnp.concatenate(ss, axis=1)
    u = _elu(jnp.dot(cat, a0_ref[...], preferred_element_type=jnp.float32)
             + c0_ref[...])
    return jnp.dot(u, a1_ref[...], preferred_element_type=jnp.float32) + c1_ref[...]


def _tc2_body(p_ref, rt_ref, bcat_ref, a0_ref, c0_ref, a1_ref, c1_ref,
              wcat_ref, h0_ref, h1_ref, h2_ref):
    st = _tc_mid(p_ref, rt_ref, bcat_ref, a0_ref, c0_ref, a1_ref, c1_ref)
    h = jnp.dot(st, wcat_ref[...], preferred_element_type=jnp.float32)
    hs = (h0_ref, h1_ref, h2_ref)
    for o in range(ORDER):
        hs[o][...] = h[:, o * D:(o + 1) * D] * rt_ref[:, o:o + 1]


def _tc3_body(p_ref, rt_ref, bcat_ref, a0_ref, c0_ref, a1_ref, c1_ref,
              out_ref):
    st = _tc_mid(p_ref, rt_ref, bcat_ref, a0_ref, c0_ref, a1_ref, c1_ref)
    m = jnp.max(st, axis=1, keepdims=True)
    e = st - m
    lse = jnp.log(jnp.sum(jnp.exp(e), axis=1, keepdims=True))
    out_ref[...] = e - lse


def _row_spec(cols):
    return pl.BlockSpec((BN, cols), lambda i: (i, 0))


def _full_spec(shape):
    nd = len(shape)
    return pl.BlockSpec(shape, lambda i, nd=nd: (0,) * nd)


_nd_f32 = jax.ShapeDtypeStruct((N, D), jnp.float32)

_tc1 = pl.pallas_call(
    _tc1_body,
    grid=(GRID,),
    in_specs=[
        _row_spec(D),
        pl.BlockSpec((NC, ORDER, BN, D), lambda i: (0, 0, i, 0)),
        _full_spec((D, ORDER * D)),
    ],
    out_specs=[_row_spec(D)] * 3 + [_row_spec(ORDER)],
    out_shape=[_nd_f32] * 3 + [jax.ShapeDtypeStruct((N, ORDER), jnp.float32)],
)

_mid_specs = [
    pl.BlockSpec((NC, ORDER, BN, D), lambda i: (0, 0, i, 0)),
    _row_spec(ORDER),
    _full_spec((1, ORDER * D)),
    _full_spec((ORDER * D, D)),
    _full_spec((1, D)),
    _full_spec((D, D)),
    _full_spec((1, D)),
]

_tc2 = pl.pallas_call(
    _tc2_body,
    grid=(GRID,),
    in_specs=_mid_specs + [_full_spec((D, ORDER * D))],
    out_specs=[_row_spec(D)] * 3,
    out_shape=[_nd_f32] * 3,
)

_tc3 = pl.pallas_call(
    _tc3_body,
    grid=(GRID,),
    in_specs=_mid_specs,
    out_specs=_row_spec(D),
    out_shape=_nd_f32,
)


def kernel(node_feature, adj0, adj1, adj2,
           W00, b00, W01, b01, W02, b02, A0_0, c0_0, A1_0, c1_0,
           W10, b10, W11, b11, W12, b12, A0_1, c0_1, A1_1, c1_1):
    srcs = [adj0[0], adj1[0], adj2[0]]
    dsts = [adj0[1], adj1[1], adj2[1]]
    onesd = jnp.ones((CHUNK, D), jnp.float32)
    zerosd = jnp.zeros((ZROWS, D), jnp.float32)

    degp = _deg_kernel()(dsts[0], dsts[1], dsts[2], onesd, zerosd)

    wcat0 = jnp.concatenate([W00, W01, W02], axis=1)
    wcat1 = jnp.concatenate([W10, W11, W12], axis=1)
    bcat0 = jnp.concatenate([b00, b01, b02]).reshape(1, ORDER * D)
    bcat1 = jnp.concatenate([b10, b11, b12]).reshape(1, ORDER * D)

    h0, h1, h2, rt = _tc1(node_feature, degp, wcat0)
    p = _spmm_kernel()(h0, h1, h2, srcs[0], dsts[0], srcs[1], dsts[1],
                     srcs[2], dsts[2], zerosd)
    h0, h1, h2 = _tc2(p, rt, bcat0, A0_0, c0_0.reshape(1, D), A1_0,
                      c1_0.reshape(1, D), wcat1)
    p = _spmm_kernel()(h0, h1, h2, srcs[0], dsts[0], srcs[1], dsts[1],
                     srcs[2], dsts[2], zerosd)
    return _tc3(p, rt, bcat1, A0_1, c0_1.reshape(1, D), A1_1,
                c1_1.reshape(1, D))


# speedup vs baseline: 16.5857x; 1.0456x over previous
import functools

import jax
import jax.numpy as jnp
from jax import lax
from jax.experimental import pallas as pl
from jax.experimental.pallas import tpu as pltpu
from jax.experimental.pallas import tpu_sc as plsc

N = 10000
E = 160000
D = 128
ORDER = 3
LANES = 16

NC = 2
NS = 16
NW = NC * NS
EPT = E // NW
CHUNK = 128
NCHK = E // CHUNK
NK = NCHK // NW
EXTRA = NCHK - NK * NW
N_PAD = 10240
RPT = N_PAD // NS
ZROWS = 64
BN = 1000
GRID = N // BN

assert E == NCHK * CHUNK and CHUNK % 8 == 0 and CHUNK <= 128
assert N_PAD == NS * RPT and RPT % ZROWS == 0 and RPT % 8 == 0

def _mesh():
    return plsc.VectorSubcoreMesh(core_axis_name="c", subcore_axis_name="s",
                                  num_cores=NC, num_subcores=NS)


def _deg_body(dst0, dst1, dst2, ones_hbm, zeros_hbm, out,
              acc, ones_b, zb, didx0, didx1, isem0, isem1, ssem0, ssem1):
    c = lax.axis_index("c")
    s = lax.axis_index("s")
    w = c * NS + s
    dsts = (dst0, dst1, dst2)
    didx = (didx0, didx1)
    isem = (isem0, isem1)
    ssem = (ssem0, ssem1)

    pltpu.sync_copy(ones_hbm, ones_b)
    pltpu.sync_copy(zeros_hbm, zb)
    for o in range(ORDER):
        dst_r = dsts[o]

        def base_of(k):
            return (k * NW + w) * CHUNK

        def issue_idx(k, b, dst_r=dst_r):
            pltpu.async_copy(dst_r.at[pl.ds(base_of(k), CHUNK)], didx[b],
                             isem[b])

        def wait_idx(k, b, dst_r=dst_r):
            pltpu.make_async_copy(dst_r.at[pl.ds(base_of(k), CHUNK)],
                                  didx[b], isem[b]).wait()

        for j in range(RPT // ZROWS):
            pltpu.sync_copy(zb, acc.at[pl.ds(s * RPT + j * ZROWS, ZROWS)])
        plsc.subcore_barrier()

        issue_idx(0, 0)

        @pl.loop(0, NK // 2)
        def _(j, issue_idx=issue_idx, wait_idx=wait_idx):
            for b in (0, 1):
                k = j * 2 + b

                @pl.when(k > 0)
                def _():
                    pltpu.make_async_copy(ones_b, acc.at[didx[1 - b]],
                                          ssem[1 - b]).wait()
                issue_idx(k + 1, 1 - b)
                wait_idx(k, b)
                pltpu.async_copy(ones_b, acc.at[didx[b]], ssem[b], add=True)

        if NK % 2 == 1:
            kl = NK - 1
            pltpu.make_async_copy(ones_b, acc.at[didx[1]], ssem[1]).wait()
            wait_idx(kl, 0)
            pltpu.async_copy(ones_b, acc.at[didx[0]], ssem[0], add=True)
        pltpu.make_async_copy(ones_b, acc.at[didx[0]], ssem[0]).wait()

        @pl.when(w < EXTRA)
        def _(dst_r=dst_r):
            base = (NK * NW + w) * CHUNK
            pltpu.sync_copy(dst_r.at[pl.ds(base, CHUNK)], didx[1])
            pltpu.sync_copy(ones_b, acc.at[didx[1]], add=True)
        plsc.subcore_barrier()

        r0 = s * RPT
        pltpu.sync_copy(acc.at[pl.ds(r0, RPT)], out.at[c, o, pl.ds(r0, RPT)])
        plsc.subcore_barrier()


def _spmm_body(h0, h1, h2, src0, dst0, src1, dst1, src2, dst2, zeros_hbm,
               out, acc, zb, sidx0, sidx1, didx0, didx1, rows0, rows1,
               isem0, isem1, gsem0, gsem1, ssem0, ssem1):
    c = lax.axis_index("c")
    s = lax.axis_index("s")
    w = c * NS + s
    hs = (h0, h1, h2)
    srcs = (src0, src1, src2)
    dsts = (dst0, dst1, dst2)
    sidx = (sidx0, sidx1)
    didx = (didx0, didx1)
    rows = (rows0, rows1)
    isem = (isem0, isem1)
    gsem = (gsem0, gsem1)
    ssem = (ssem0, ssem1)

    pltpu.sync_copy(zeros_hbm, zb)
    for o in range(ORDER):
        src_r, dst_r, h_r = srcs[o], dsts[o], hs[o]

        def base_of(k):
            return (k * NW + w) * CHUNK

        def issue_idx(k, b, src_r=src_r, dst_r=dst_r):
            pltpu.async_copy(src_r.at[pl.ds(base_of(k), CHUNK)], sidx[b],
                             isem[b])
            pltpu.async_copy(dst_r.at[pl.ds(base_of(k), CHUNK)], didx[b],
                             isem[b])

        def wait_idx(k, b, src_r=src_r, dst_r=dst_r):
            pltpu.make_async_copy(src_r.at[pl.ds(base_of(k), CHUNK)],
                                  sidx[b], isem[b]).wait()
            pltpu.make_async_copy(dst_r.at[pl.ds(base_of(k), CHUNK)],
                                  didx[b], isem[b]).wait()

        for j in range(RPT // ZROWS):
            pltpu.sync_copy(zb, acc.at[pl.ds(s * RPT + j * ZROWS, ZROWS)])
        plsc.subcore_barrier()

        issue_idx(0, 0)

        @pl.loop(0, NK // 2)
        def _(j, issue_idx=issue_idx, wait_idx=wait_idx, h_r=h_r):
            for b in (0, 1):
                k = j * 2 + b
                wait_idx(k, b)
                pltpu.async_copy(h_r.at[sidx[b]], rows[b], gsem[b])

                @pl.when(k > 0)
                def _():
                    pltpu.make_async_copy(rows[1 - b], acc.at[didx[1 - b]],
                                          ssem[1 - b]).wait()
                issue_idx(k + 1, 1 - b)
                pltpu.make_async_copy(h_r.at[sidx[b]], rows[b],
                                      gsem[b]).wait()
                pltpu.async_copy(rows[b], acc.at[didx[b]], ssem[b], add=True)

        if NK % 2 == 1:
            kl = NK - 1
            wait_idx(kl, 0)
            pltpu.async_copy(h_r.at[sidx[0]], rows[0], gsem[0])
            pltpu.make_async_copy(rows[1], acc.at[didx[1]], ssem[1]).wait()
            pltpu.make_async_copy(h_r.at[sidx[0]], rows[0], gsem[0]).wait()
            pltpu.async_copy(rows[0], acc.at[didx[0]], ssem[0], add=True)
        pltpu.make_async_copy(rows[0], acc.at[didx[0]], ssem[0]).wait()

        @pl.when(w < EXTRA)
        def _(src_r=src_r, dst_r=dst_r, h_r=h_r):
            base = (NK * NW + w) * CHUNK
            pltpu.sync_copy(src_r.at[pl.ds(base, CHUNK)], sidx[1])
            pltpu.sync_copy(dst_r.at[pl.ds(base, CHUNK)], didx[1])
            pltpu.async_copy(h_r.at[sidx[1]], rows[1], gsem[1]).wait()
            pltpu.sync_copy(rows[1], acc.at[didx[1]], add=True)
        plsc.subcore_barrier()

        r0 = s * RPT
        pltpu.sync_copy(acc.at[pl.ds(r0, RPT)], out.at[c, o, pl.ds(r0, RPT)])
        plsc.subcore_barrier()


@functools.cache
def _deg_kernel():
    return pl.kernel(
        _deg_body,
        out_type=jax.ShapeDtypeStruct((NC, ORDER, N_PAD, D), jnp.float32),
        mesh=_mesh(),
        scratch_types=[
            pltpu.VMEM_SHARED((N_PAD, D), jnp.float32),
            pltpu.VMEM((CHUNK, D), jnp.float32),
            pltpu.VMEM((ZROWS, D), jnp.float32),
            pltpu.VMEM((CHUNK,), jnp.int32),
            pltpu.VMEM((CHUNK,), jnp.int32),
            pltpu.SemaphoreType.DMA,
            pltpu.SemaphoreType.DMA,
            pltpu.SemaphoreType.DMA,
            pltpu.SemaphoreType.DMA,
        ],
    )


@functools.cache
def _spmm_kernel():
    return pl.kernel(
        _spmm_body,
        out_type=jax.ShapeDtypeStruct((NC, ORDER, N_PAD, D), jnp.float32),
        mesh=_mesh(),
        scratch_types=[
            pltpu.VMEM_SHARED((N_PAD, D), jnp.float32),
            pltpu.VMEM((ZROWS, D), jnp.float32),
            pltpu.VMEM((CHUNK,), jnp.int32),
            pltpu.VMEM((CHUNK,), jnp.int32),
            pltpu.VMEM((CHUNK,), jnp.int32),
            pltpu.VMEM((CHUNK,), jnp.int32),
            pltpu.VMEM((CHUNK, D), jnp.float32),
            pltpu.VMEM((CHUNK, D), jnp.float32),
            pltpu.SemaphoreType.DMA,
            pltpu.SemaphoreType.DMA,
            pltpu.SemaphoreType.DMA,
            pltpu.SemaphoreType.DMA,
            pltpu.SemaphoreType.DMA,
            pltpu.SemaphoreType.DMA,
        ],
    )


def _elu(x):
    return jnp.where(x > 0, x, jnp.exp(x) - 1.0)


def _tc1_body(x_ref, degp_ref, wcat_ref, h0_ref, h1_ref, h2_ref, rt_ref):
    h = jnp.dot(x_ref[...], wcat_ref[...], preferred_element_type=jnp.float32)
    hs = (h0_ref, h1_ref, h2_ref)
    for o in range(ORDER):
        deg = degp_ref[0, o, :, 0:1] + degp_ref[1, o, :, 0:1]
        r = lax.rsqrt(jnp.maximum(deg, 1.0))
        hs[o][...] = h[:, o * D:(o + 1) * D] * r
        rt_ref[:, o:o + 1] = r


def _tc_mid(p_ref, rt_ref, bcat_ref, a0_ref, c0_ref, a1_ref, c1_ref):
    ss = []
    for o in range(ORDER):
        r = rt_ref[:, o:o + 1]
        t = (p_ref[0, o] + p_ref[1, o]) * r + bcat_ref[:, o * D:(o + 1) * D]
        ss.append(_elu(t))
    cat = jnp.concatenate(ss, axis=1)
    u = _elu(jnp.dot(cat, a0_ref[...], preferred_element_type=jnp.float32)
             + c0_ref[...])
    return jnp.dot(u, a1_ref[...], preferred_element_type=jnp.float32) + c1_ref[...]


def _tc2_body(p_ref, rt_ref, bcat_ref, a0_ref, c0_ref, a1_ref, c1_ref,
              wcat_ref, h0_ref, h1_ref, h2_ref):
    st = _tc_mid(p_ref, rt_ref, bcat_ref, a0_ref, c0_ref, a1_ref, c1_ref)
    h = jnp.dot(st, wcat_ref[...], preferred_element_type=jnp.float32)
    hs = (h0_ref, h1_ref, h2_ref)
    for o in range(ORDER):
        hs[o][...] = h[:, o * D:(o + 1) * D] * rt_ref[:, o:o + 1]


def _tc3_body(p_ref, rt_ref, bcat_ref, a0_ref, c0_ref, a1_ref, c1_ref,
              out_ref):
    st = _tc_mid(p_ref, rt_ref, bcat_ref, a0_ref, c0_ref, a1_ref, c1_ref)
    m = jnp.max(st, axis=1, keepdims=True)
    e = st - m
    lse = jnp.log(jnp.sum(jnp.exp(e), axis=1, keepdims=True))
    out_ref[...] = e - lse


def _row_spec(cols):
    return pl.BlockSpec((BN, cols), lambda i: (i, 0))


def _full_spec(shape):
    nd = len(shape)
    return pl.BlockSpec(shape, lambda i, nd=nd: (0,) * nd)


_nd_f32 = jax.ShapeDtypeStruct((N, D), jnp.float32)

_tc1 = pl.pallas_call(
    _tc1_body,
    grid=(GRID,),
    in_specs=[
        _row_spec(D),
        pl.BlockSpec((NC, ORDER, BN, D), lambda i: (0, 0, i, 0)),
        _full_spec((D, ORDER * D)),
    ],
    out_specs=[_row_spec(D)] * 3 + [_row_spec(ORDER)],
    out_shape=[_nd_f32] * 3 + [jax.ShapeDtypeStruct((N, ORDER), jnp.float32)],
)

_mid_specs = [
    pl.BlockSpec((NC, ORDER, BN, D), lambda i: (0, 0, i, 0)),
    _row_spec(ORDER),
    _full_spec((1, ORDER * D)),
    _full_spec((ORDER * D, D)),
    _full_spec((1, D)),
    _full_spec((D, D)),
    _full_spec((1, D)),
]

_tc2 = pl.pallas_call(
    _tc2_body,
    grid=(GRID,),
    in_specs=_mid_specs + [_full_spec((D, ORDER * D))],
    out_specs=[_row_spec(D)] * 3,
    out_shape=[_nd_f32] * 3,
)

_tc3 = pl.pallas_call(
    _tc3_body,
    grid=(GRID,),
    in_specs=_mid_specs,
    out_specs=_row_spec(D),
    out_shape=_nd_f32,
)


def kernel(node_feature, adj0, adj1, adj2,
           W00, b00, W01, b01, W02, b02, A0_0, c0_0, A1_0, c1_0,
           W10, b10, W11, b11, W12, b12, A0_1, c0_1, A1_1, c1_1):
    srcs = [adj0[0], adj1[0], adj2[0]]
    dsts = [adj0[1], adj1[1], adj2[1]]
    onesd = jnp.ones((CHUNK, D), jnp.float32)
    zerosd = jnp.zeros((ZROWS, D), jnp.float32)

    degp = _deg_kernel()(dsts[0], dsts[1], dsts[2], onesd, zerosd)

    wcat0 = jnp.concatenate([W00, W01, W02], axis=1)
    wcat1 = jnp.concatenate([W10, W11, W12], axis=1)
    bcat0 = jnp.concatenate([b00, b01, b02]).reshape(1, ORDER * D)
    bcat1 = jnp.concatenate([b10, b11, b12]).reshape(1, ORDER * D)

    h0, h1, h2, rt = _tc1(node_feature, degp, wcat0)
    p = _spmm_kernel()(h0, h1, h2, srcs[0], dsts[0], srcs[1], dsts[1],
                     srcs[2], dsts[2], zerosd)
    h0, h1, h2 = _tc2(p, rt, bcat0, A0_0, c0_0.reshape(1, D), A1_0,
                      c1_0.reshape(1, D), wcat1)
    p = _spmm_kernel()(h0, h1, h2, srcs[0], dsts[0], srcs[1], dsts[1],
                     srcs[2], dsts[2], zerosd)
    return _tc3(p, rt, bcat1, A0_1, c0_1.reshape(1, D), A1_1,
                c1_1.reshape(1, D))


# speedup vs baseline: 18.3322x; 1.1053x over previous
import functools

import jax
import jax.numpy as jnp
from jax import lax
from jax.experimental import pallas as pl
from jax.experimental.pallas import tpu as pltpu
from jax.experimental.pallas import tpu_sc as plsc

N = 10000
E = 160000
D = 128
ORDER = 3
LANES = 16

NC = 2
NS = 16
NW = NC * NS
EPT = E // NW
CHUNK = 128
NCHK = E // CHUNK
NK = NCHK // NW
EXTRA = NCHK - NK * NW
N_PAD = 10240
RPT = N_PAD // NS
ZROWS = 64
BN = 1000
GRID = N // BN

assert E == NCHK * CHUNK and CHUNK % 8 == 0 and CHUNK <= 128
assert N_PAD == NS * RPT and RPT % ZROWS == 0 and RPT % 8 == 0

def _mesh():
    return plsc.VectorSubcoreMesh(core_axis_name="c", subcore_axis_name="s",
                                  num_cores=NC, num_subcores=NS)


def _deg_body(dst0, dst1, dst2, ones_hbm, zeros_hbm, out,
              acc, ones_b, zb, didx0, didx1, isem0, isem1, ssem0, ssem1):
    c = lax.axis_index("c")
    s = lax.axis_index("s")
    w = c * NS + s
    dsts = (dst0, dst1, dst2)
    didx = (didx0, didx1)
    isem = (isem0, isem1)
    ssem = (ssem0, ssem1)

    pltpu.sync_copy(ones_hbm, ones_b)
    pltpu.sync_copy(zeros_hbm, zb)
    for o in range(ORDER):
        dst_r = dsts[o]

        def base_of(k):
            return (k * NW + w) * CHUNK

        def issue_idx(k, b, dst_r=dst_r):
            pltpu.async_copy(dst_r.at[pl.ds(base_of(k), CHUNK)], didx[b],
                             isem[b])

        def wait_idx(k, b, dst_r=dst_r):
            pltpu.make_async_copy(dst_r.at[pl.ds(base_of(k), CHUNK)],
                                  didx[b], isem[b]).wait()

        for j in range(RPT // ZROWS):
            pltpu.sync_copy(zb, acc.at[pl.ds(s * RPT + j * ZROWS, ZROWS)])
        plsc.subcore_barrier()

        issue_idx(0, 0)

        @pl.loop(0, NK // 2)
        def _(j, issue_idx=issue_idx, wait_idx=wait_idx):
            for b in (0, 1):
                k = j * 2 + b

                @pl.when(k > 0)
                def _():
                    pltpu.make_async_copy(ones_b, acc.at[didx[1 - b]],
                                          ssem[1 - b]).wait()
                issue_idx(k + 1, 1 - b)
                wait_idx(k, b)
                pltpu.async_copy(ones_b, acc.at[didx[b]], ssem[b], add=True)

        if NK % 2 == 1:
            kl = NK - 1
            pltpu.make_async_copy(ones_b, acc.at[didx[1]], ssem[1]).wait()
            wait_idx(kl, 0)
            pltpu.async_copy(ones_b, acc.at[didx[0]], ssem[0], add=True)
        pltpu.make_async_copy(ones_b, acc.at[didx[0]], ssem[0]).wait()

        @pl.when(w < EXTRA)
        def _(dst_r=dst_r):
            base = (NK * NW + w) * CHUNK
            pltpu.sync_copy(dst_r.at[pl.ds(base, CHUNK)], didx[1])
            pltpu.sync_copy(ones_b, acc.at[didx[1]], add=True)
        plsc.subcore_barrier()

        r0 = s * RPT
        pltpu.sync_copy(acc.at[pl.ds(r0, RPT)], out.at[c, o, pl.ds(r0, RPT)])
        plsc.subcore_barrier()


def _spmm_body(h0, h1, h2, src0, dst0, src1, dst1, src2, dst2, zeros_hbm,
               out, acc, zb, sidx0, sidx1, sidx2, didx0, didx1, didx2,
               rows0, rows1, isem0, isem1, isem2, gsem0, gsem1, ssem0, ssem1):
    c = lax.axis_index("c")
    s = lax.axis_index("s")
    w = c * NS + s
    hs = (h0, h1, h2)
    srcs = (src0, src1, src2)
    dsts = (dst0, dst1, dst2)
    sidx = (sidx0, sidx1, sidx2)
    didx = (didx0, didx1, didx2)
    rows = (rows0, rows1)
    isem = (isem0, isem1, isem2)
    gsem = (gsem0, gsem1)
    ssem = (ssem0, ssem1)

    pltpu.sync_copy(zeros_hbm, zb)
    for j in range(RPT // ZROWS):
        pltpu.sync_copy(zb, acc.at[pl.ds(s * RPT + j * ZROWS, ZROWS)])
    plsc.subcore_barrier()

    for o in range(ORDER):
        src_r, dst_r, h_r = srcs[o], dsts[o], hs[o]

        def base_of(k):
            return (k * NW + w) * CHUNK

        def issue_idx(k, b, src_r=src_r, dst_r=dst_r):
            pltpu.async_copy(src_r.at[pl.ds(base_of(k), CHUNK)], sidx[b],
                             isem[b])
            pltpu.async_copy(dst_r.at[pl.ds(base_of(k), CHUNK)], didx[b],
                             isem[b])

        def wait_idx(k, b, src_r=src_r, dst_r=dst_r):
            pltpu.make_async_copy(src_r.at[pl.ds(base_of(k), CHUNK)],
                                  sidx[b], isem[b]).wait()
            pltpu.make_async_copy(dst_r.at[pl.ds(base_of(k), CHUNK)],
                                  didx[b], isem[b]).wait()

        def issue_gather(b3, b2, h_r=h_r):
            pltpu.async_copy(h_r.at[sidx[b3]], rows[b2], gsem[b2])

        def wait_gather(b3, b2, h_r=h_r):
            pltpu.make_async_copy(h_r.at[sidx[b3]], rows[b2],
                                  gsem[b2]).wait()

        def issue_scatter(b3, b2):
            pltpu.async_copy(rows[b2], acc.at[didx[b3]], ssem[b2], add=True)

        def wait_scatter(b3, b2):
            pltpu.make_async_copy(rows[b2], acc.at[didx[b3]],
                                  ssem[b2]).wait()

        def step(k, kmod, first, prefetch):
            b2 = kmod % 2
            b3 = kmod % 3
            if not first:
                wait_scatter((kmod - 1) % 3, 1 - b2)
            wait_idx(k + 1, (kmod + 1) % 3)
            issue_gather((kmod + 1) % 3, 1 - b2)
            wait_gather(b3, b2)
            issue_scatter(b3, b2)
            if prefetch:
                issue_idx(k + 2, (kmod + 2) % 3)

        issue_idx(0, 0)
        issue_idx(1, 1)
        wait_idx(0, 0)
        issue_gather(0, 0)

        step(0, 0, True, 2 <= NK - 1)
        nmain = ((NK - 2) // 6) * 6

        @pl.loop(0, nmain // 6)
        def _(j6, step=step):
            for i in range(6):
                k = j6 * 6 + 1 + i
                step(k, (1 + i) % 6, False, True)

        for k in range(1 + nmain, NK - 1):
            step(k, k % 6, False, k + 2 <= NK - 1)

        kl = NK - 1
        wait_scatter((kl - 1) % 3, 1 - (kl % 2))
        wait_gather(kl % 3, kl % 2)
        issue_scatter(kl % 3, kl % 2)
        wait_scatter(kl % 3, kl % 2)

        @pl.when(w < EXTRA)
        def _(src_r=src_r, dst_r=dst_r, h_r=h_r):
            base = (NK * NW + w) * CHUNK
            pltpu.sync_copy(src_r.at[pl.ds(base, CHUNK)], sidx[0])
            pltpu.sync_copy(dst_r.at[pl.ds(base, CHUNK)], didx[0])
            pltpu.async_copy(h_r.at[sidx[0]], rows[0], gsem[0]).wait()
            pltpu.sync_copy(rows[0], acc.at[didx[0]], add=True)
        plsc.subcore_barrier()

        r0 = s * RPT
        pltpu.sync_copy(acc.at[pl.ds(r0, RPT)], out.at[c, o, pl.ds(r0, RPT)])
        if o + 1 < ORDER:
            for j in range(RPT // ZROWS):
                pltpu.sync_copy(zb, acc.at[pl.ds(r0 + j * ZROWS, ZROWS)])
        plsc.subcore_barrier()


@functools.cache
def _deg_kernel():
    return pl.kernel(
        _deg_body,
        out_type=jax.ShapeDtypeStruct((NC, ORDER, N_PAD, D), jnp.float32),
        mesh=_mesh(),
        scratch_types=[
            pltpu.VMEM_SHARED((N_PAD, D), jnp.float32),
            pltpu.VMEM((CHUNK, D), jnp.float32),
            pltpu.VMEM((ZROWS, D), jnp.float32),
            pltpu.VMEM((CHUNK,), jnp.int32),
            pltpu.VMEM((CHUNK,), jnp.int32),
            pltpu.SemaphoreType.DMA,
            pltpu.SemaphoreType.DMA,
            pltpu.SemaphoreType.DMA,
            pltpu.SemaphoreType.DMA,
        ],
    )


@functools.cache
def _spmm_kernel():
    return pl.kernel(
        _spmm_body,
        out_type=jax.ShapeDtypeStruct((NC, ORDER, N_PAD, D), jnp.float32),
        mesh=_mesh(),
        scratch_types=[
            pltpu.VMEM_SHARED((N_PAD, D), jnp.float32),
            pltpu.VMEM((ZROWS, D), jnp.float32),
            pltpu.VMEM((CHUNK,), jnp.int32),
            pltpu.VMEM((CHUNK,), jnp.int32),
            pltpu.VMEM((CHUNK,), jnp.int32),
            pltpu.VMEM((CHUNK,), jnp.int32),
            pltpu.VMEM((CHUNK,), jnp.int32),
            pltpu.VMEM((CHUNK,), jnp.int32),
            pltpu.VMEM((CHUNK, D), jnp.float32),
            pltpu.VMEM((CHUNK, D), jnp.float32),
            pltpu.SemaphoreType.DMA,
            pltpu.SemaphoreType.DMA,
            pltpu.SemaphoreType.DMA,
            pltpu.SemaphoreType.DMA,
            pltpu.SemaphoreType.DMA,
            pltpu.SemaphoreType.DMA,
            pltpu.SemaphoreType.DMA,
        ],
    )


def _elu(x):
    return jnp.where(x > 0, x, jnp.exp(x) - 1.0)


def _tc1_body(x_ref, degp_ref, wcat_ref, h0_ref, h1_ref, h2_ref, rt_ref):
    h = jnp.dot(x_ref[...], wcat_ref[...], preferred_element_type=jnp.float32)
    hs = (h0_ref, h1_ref, h2_ref)
    for o in range(ORDER):
        deg = degp_ref[0, o, :, 0:1] + degp_ref[1, o, :, 0:1]
        r = lax.rsqrt(jnp.maximum(deg, 1.0))
        hs[o][...] = h[:, o * D:(o + 1) * D] * r
        rt_ref[:, o:o + 1] = r


def _tc_mid(p_ref, rt_ref, bcat_ref, a0_ref, c0_ref, a1_ref, c1_ref):
    ss = []
    for o in range(ORDER):
        r = rt_ref[:, o:o + 1]
        t = (p_ref[0, o] + p_ref[1, o]) * r + bcat_ref[:, o * D:(o + 1) * D]
        ss.append(_elu(t))
    cat = jnp.concatenate(ss, axis=1)
    u = _elu(jnp.dot(cat, a0_ref[...], preferred_element_type=jnp.float32)
             + c0_ref[...])
    return jnp.dot(u, a1_ref[...], preferred_element_type=jnp.float32) + c1_ref[...]


def _tc2_body(p_ref, rt_ref, bcat_ref, a0_ref, c0_ref, a1_ref, c1_ref,
              wcat_ref, h0_ref, h1_ref, h2_ref):
    st = _tc_mid(p_ref, rt_ref, bcat_ref, a0_ref, c0_ref, a1_ref, c1_ref)
    h = jnp.dot(st, wcat_ref[...], preferred_element_type=jnp.float32)
    hs = (h0_ref, h1_ref, h2_ref)
    for o in range(ORDER):
        hs[o][...] = h[:, o * D:(o + 1) * D] * rt_ref[:, o:o + 1]


def _tc3_body(p_ref, rt_ref, bcat_ref, a0_ref, c0_ref, a1_ref, c1_ref,
              out_ref):
    st = _tc_mid(p_ref, rt_ref, bcat_ref, a0_ref, c0_ref, a1_ref, c1_ref)
    m = jnp.max(st, axis=1, keepdims=True)
    e = st - m
    lse = jnp.log(jnp.sum(jnp.exp(e), axis=1, keepdims=True))
    out_ref[...] = e - lse


def _row_spec(cols):
    return pl.BlockSpec((BN, cols), lambda i: (i, 0))


def _full_spec(shape):
    nd = len(shape)
    return pl.BlockSpec(shape, lambda i, nd=nd: (0,) * nd)


_nd_f32 = jax.ShapeDtypeStruct((N, D), jnp.float32)

_tc1 = pl.pallas_call(
    _tc1_body,
    grid=(GRID,),
    in_specs=[
        _row_spec(D),
        pl.BlockSpec((NC, ORDER, BN, D), lambda i: (0, 0, i, 0)),
        _full_spec((D, ORDER * D)),
    ],
    out_specs=[_row_spec(D)] * 3 + [_row_spec(ORDER)],
    out_shape=[_nd_f32] * 3 + [jax.ShapeDtypeStruct((N, ORDER), jnp.float32)],
)

_mid_specs = [
    pl.BlockSpec((NC, ORDER, BN, D), lambda i: (0, 0, i, 0)),
    _row_spec(ORDER),
    _full_spec((1, ORDER * D)),
    _full_spec((ORDER * D, D)),
    _full_spec((1, D)),
    _full_spec((D, D)),
    _full_spec((1, D)),
]

_tc2 = pl.pallas_call(
    _tc2_body,
    grid=(GRID,),
    in_specs=_mid_specs + [_full_spec((D, ORDER * D))],
    out_specs=[_row_spec(D)] * 3,
    out_shape=[_nd_f32] * 3,
)

_tc3 = pl.pallas_call(
    _tc3_body,
    grid=(GRID,),
    in_specs=_mid_specs,
    out_specs=_row_spec(D),
    out_shape=_nd_f32,
)


def kernel(node_feature, adj0, adj1, adj2,
           W00, b00, W01, b01, W02, b02, A0_0, c0_0, A1_0, c1_0,
           W10, b10, W11, b11, W12, b12, A0_1, c0_1, A1_1, c1_1):
    srcs = [adj0[0], adj1[0], adj2[0]]
    dsts = [adj0[1], adj1[1], adj2[1]]
    onesd = jnp.ones((CHUNK, D), jnp.float32)
    zerosd = jnp.zeros((ZROWS, D), jnp.float32)

    degp = _deg_kernel()(dsts[0], dsts[1], dsts[2], onesd, zerosd)

    wcat0 = jnp.concatenate([W00, W01, W02], axis=1)
    wcat1 = jnp.concatenate([W10, W11, W12], axis=1)
    bcat0 = jnp.concatenate([b00, b01, b02]).reshape(1, ORDER * D)
    bcat1 = jnp.concatenate([b10, b11, b12]).reshape(1, ORDER * D)

    h0, h1, h2, rt = _tc1(node_feature, degp, wcat0)
    p = _spmm_kernel()(h0, h1, h2, srcs[0], dsts[0], srcs[1], dsts[1],
                     srcs[2], dsts[2], zerosd)
    h0, h1, h2 = _tc2(p, rt, bcat0, A0_0, c0_0.reshape(1, D), A1_0,
                      c1_0.reshape(1, D), wcat1)
    p = _spmm_kernel()(h0, h1, h2, srcs[0], dsts[0], srcs[1], dsts[1],
                     srcs[2], dsts[2], zerosd)
    return _tc3(p, rt, bcat1, A0_1, c0_1.reshape(1, D), A1_1,
                c1_1.reshape(1, D))


# speedup vs baseline: 18.4694x; 1.0075x over previous
import functools

import jax
import jax.numpy as jnp
from jax import lax
from jax.experimental import pallas as pl
from jax.experimental.pallas import tpu as pltpu
from jax.experimental.pallas import tpu_sc as plsc

N = 10000
E = 160000
D = 128
ORDER = 3
LANES = 16

NC = 2
NS = 16
NW = NC * NS
EPT = E // NW
CHUNK = 128
NCHK = E // CHUNK
NK = NCHK // NW
EXTRA = NCHK - NK * NW
N_PAD = 10240
RPT = N_PAD // NS
ZROWS = 64
BN = 1000
GRID = N // BN

assert E == NCHK * CHUNK and CHUNK % 8 == 0 and CHUNK <= 128
assert N_PAD == NS * RPT and RPT % ZROWS == 0 and RPT % 8 == 0

def _mesh():
    return plsc.VectorSubcoreMesh(core_axis_name="c", subcore_axis_name="s",
                                  num_cores=NC, num_subcores=NS)


def _deg_body(dst0, dst1, dst2, ones_hbm, zeros_hbm, out,
              acc, ones_b, zb, didx0, didx1, didx2, didx3,
              isem0, isem1, isem2, isem3, ssem0, ssem1):
    c = lax.axis_index("c")
    s = lax.axis_index("s")
    w = c * NS + s
    dsts = (dst0, dst1, dst2)
    didx = (didx0, didx1, didx2, didx3)
    isem = (isem0, isem1, isem2, isem3)
    ssem = (ssem0, ssem1)

    pltpu.sync_copy(ones_hbm, ones_b)
    pltpu.sync_copy(zeros_hbm, zb)
    for j in range(RPT // ZROWS):
        pltpu.sync_copy(zb, acc.at[pl.ds(s * RPT + j * ZROWS, ZROWS)])
    plsc.subcore_barrier()

    for o in range(ORDER):
        dst_r = dsts[o]

        def base_of(k):
            return (k * NW + w) * CHUNK

        def issue_idx(k, b, dst_r=dst_r):
            pltpu.async_copy(dst_r.at[pl.ds(base_of(k), CHUNK)], didx[b],
                             isem[b])

        def wait_idx(k, b, dst_r=dst_r):
            pltpu.make_async_copy(dst_r.at[pl.ds(base_of(k), CHUNK)],
                                  didx[b], isem[b]).wait()

        def wait_scatter(b4, b2):
            pltpu.make_async_copy(ones_b, acc.at[didx[b4]], ssem[b2]).wait()

        def step(k, kmod, first, prefetch):
            if not first:
                wait_scatter((kmod - 2) % 4, kmod % 2)
            wait_idx(k, kmod % 4)
            pltpu.async_copy(ones_b, acc.at[didx[kmod % 4]], ssem[kmod % 2],
                             add=True)
            if prefetch:
                issue_idx(k + 2, (kmod + 2) % 4)

        issue_idx(0, 0)
        issue_idx(1, 1)
        step(0, 0, True, 2 <= NK - 1)
        step(1, 1, True, 3 <= NK - 1)
        nmain = ((NK - 2) // 4) * 4

        @pl.loop(0, nmain // 4)
        def _(j4, step=step):
            for i in range(4):
                k = j4 * 4 + 2 + i
                step(k, (2 + i) % 4, False, True)

        for k in range(2 + nmain, NK):
            step(k, k % 4, False, k + 2 <= NK - 1)

        wait_scatter((NK - 2) % 4, (NK - 2) % 2)
        wait_scatter((NK - 1) % 4, (NK - 1) % 2)

        @pl.when(w < EXTRA)
        def _(dst_r=dst_r):
            base = (NK * NW + w) * CHUNK
            pltpu.sync_copy(dst_r.at[pl.ds(base, CHUNK)], didx[0])
            pltpu.sync_copy(ones_b, acc.at[didx[0]], add=True)
        plsc.subcore_barrier()

        r0 = s * RPT
        pltpu.sync_copy(acc.at[pl.ds(r0, RPT)], out.at[c, o, pl.ds(r0, RPT)])
        if o + 1 < ORDER:
            for j in range(RPT // ZROWS):
                pltpu.sync_copy(zb, acc.at[pl.ds(r0 + j * ZROWS, ZROWS)])
        plsc.subcore_barrier()


def _spmm_body(h0, h1, h2, src0, dst0, src1, dst1, src2, dst2, zeros_hbm,
               out, acc, zb, sidx0, sidx1, sidx2, didx0, didx1, didx2,
               rows0, rows1, isem0, isem1, isem2, gsem0, gsem1, ssem0, ssem1):
    c = lax.axis_index("c")
    s = lax.axis_index("s")
    w = c * NS + s
    hs = (h0, h1, h2)
    srcs = (src0, src1, src2)
    dsts = (dst0, dst1, dst2)
    sidx = (sidx0, sidx1, sidx2)
    didx = (didx0, didx1, didx2)
    rows = (rows0, rows1)
    isem = (isem0, isem1, isem2)
    gsem = (gsem0, gsem1)
    ssem = (ssem0, ssem1)

    pltpu.sync_copy(zeros_hbm, zb)
    for j in range(RPT // ZROWS):
        pltpu.sync_copy(zb, acc.at[pl.ds(s * RPT + j * ZROWS, ZROWS)])
    plsc.subcore_barrier()

    for o in range(ORDER):
        src_r, dst_r, h_r = srcs[o], dsts[o], hs[o]

        def base_of(k):
            return (k * NW + w) * CHUNK

        def issue_idx(k, b, src_r=src_r, dst_r=dst_r):
            pltpu.async_copy(src_r.at[pl.ds(base_of(k), CHUNK)], sidx[b],
                             isem[b])
            pltpu.async_copy(dst_r.at[pl.ds(base_of(k), CHUNK)], didx[b],
                             isem[b])

        def wait_idx(k, b, src_r=src_r, dst_r=dst_r):
            pltpu.make_async_copy(src_r.at[pl.ds(base_of(k), CHUNK)],
                                  sidx[b], isem[b]).wait()
            pltpu.make_async_copy(dst_r.at[pl.ds(base_of(k), CHUNK)],
                                  didx[b], isem[b]).wait()

        def issue_gather(b3, b2, h_r=h_r):
            pltpu.async_copy(h_r.at[sidx[b3]], rows[b2], gsem[b2])

        def wait_gather(b3, b2, h_r=h_r):
            pltpu.make_async_copy(h_r.at[sidx[b3]], rows[b2],
                                  gsem[b2]).wait()

        def issue_scatter(b3, b2):
            pltpu.async_copy(rows[b2], acc.at[didx[b3]], ssem[b2], add=True)

        def wait_scatter(b3, b2):
            pltpu.make_async_copy(rows[b2], acc.at[didx[b3]],
                                  ssem[b2]).wait()

        def step(k, kmod, first, prefetch):
            b2 = kmod % 2
            b3 = kmod % 3
            if not first:
                wait_scatter((kmod - 1) % 3, 1 - b2)
            wait_idx(k + 1, (kmod + 1) % 3)
            issue_gather((kmod + 1) % 3, 1 - b2)
            wait_gather(b3, b2)
            issue_scatter(b3, b2)
            if prefetch:
                issue_idx(k + 2, (kmod + 2) % 3)

        issue_idx(0, 0)
        issue_idx(1, 1)
        wait_idx(0, 0)
        issue_gather(0, 0)

        step(0, 0, True, 2 <= NK - 1)
        nmain = ((NK - 2) // 6) * 6

        @pl.loop(0, nmain // 6)
        def _(j6, step=step):
            for i in range(6):
                k = j6 * 6 + 1 + i
                step(k, (1 + i) % 6, False, True)

        for k in range(1 + nmain, NK - 1):
            step(k, k % 6, False, k + 2 <= NK - 1)

        kl = NK - 1
        wait_scatter((kl - 1) % 3, 1 - (kl % 2))
        wait_gather(kl % 3, kl % 2)
        issue_scatter(kl % 3, kl % 2)
        wait_scatter(kl % 3, kl % 2)

        @pl.when(w < EXTRA)
        def _(src_r=src_r, dst_r=dst_r, h_r=h_r):
            base = (NK * NW + w) * CHUNK
            pltpu.sync_copy(src_r.at[pl.ds(base, CHUNK)], sidx[0])
            pltpu.sync_copy(dst_r.at[pl.ds(base, CHUNK)], didx[0])
            pltpu.async_copy(h_r.at[sidx[0]], rows[0], gsem[0]).wait()
            pltpu.sync_copy(rows[0], acc.at[didx[0]], add=True)
        plsc.subcore_barrier()

        r0 = s * RPT
        pltpu.sync_copy(acc.at[pl.ds(r0, RPT)], out.at[c, o, pl.ds(r0, RPT)])
        if o + 1 < ORDER:
            for j in range(RPT // ZROWS):
                pltpu.sync_copy(zb, acc.at[pl.ds(r0 + j * ZROWS, ZROWS)])
        plsc.subcore_barrier()


@functools.cache
def _deg_kernel():
    return pl.kernel(
        _deg_body,
        out_type=jax.ShapeDtypeStruct((NC, ORDER, N_PAD, D), jnp.float32),
        mesh=_mesh(),
        scratch_types=[
            pltpu.VMEM_SHARED((N_PAD, D), jnp.float32),
            pltpu.VMEM((CHUNK, D), jnp.float32),
            pltpu.VMEM((ZROWS, D), jnp.float32),
            pltpu.VMEM((CHUNK,), jnp.int32),
            pltpu.VMEM((CHUNK,), jnp.int32),
            pltpu.VMEM((CHUNK,), jnp.int32),
            pltpu.VMEM((CHUNK,), jnp.int32),
            pltpu.SemaphoreType.DMA,
            pltpu.SemaphoreType.DMA,
            pltpu.SemaphoreType.DMA,
            pltpu.SemaphoreType.DMA,
            pltpu.SemaphoreType.DMA,
            pltpu.SemaphoreType.DMA,
        ],
    )


@functools.cache
def _spmm_kernel():
    return pl.kernel(
        _spmm_body,
        out_type=jax.ShapeDtypeStruct((NC, ORDER, N_PAD, D), jnp.float32),
        mesh=_mesh(),
        scratch_types=[
            pltpu.VMEM_SHARED((N_PAD, D), jnp.float32),
            pltpu.VMEM((ZROWS, D), jnp.float32),
            pltpu.VMEM((CHUNK,), jnp.int32),
            pltpu.VMEM((CHUNK,), jnp.int32),
            pltpu.VMEM((CHUNK,), jnp.int32),
            pltpu.VMEM((CHUNK,), jnp.int32),
            pltpu.VMEM((CHUNK,), jnp.int32),
            pltpu.VMEM((CHUNK,), jnp.int32),
            pltpu.VMEM((CHUNK, D), jnp.float32),
            pltpu.VMEM((CHUNK, D), jnp.float32),
            pltpu.SemaphoreType.DMA,
            pltpu.SemaphoreType.DMA,
            pltpu.SemaphoreType.DMA,
            pltpu.SemaphoreType.DMA,
            pltpu.SemaphoreType.DMA,
            pltpu.SemaphoreType.DMA,
            pltpu.SemaphoreType.DMA,
        ],
    )


def _elu(x):
    return jnp.where(x > 0, x, jnp.exp(x) - 1.0)


def _tc1_body(x_ref, degp_ref, wcat_ref, h0_ref, h1_ref, h2_ref, rt_ref):
    h = jnp.dot(x_ref[...], wcat_ref[...], preferred_element_type=jnp.float32)
    hs = (h0_ref, h1_ref, h2_ref)
    for o in range(ORDER):
        deg = degp_ref[0, o, :, 0:1] + degp_ref[1, o, :, 0:1]
        r = lax.rsqrt(jnp.maximum(deg, 1.0))
        hs[o][...] = h[:, o * D:(o + 1) * D] * r
        rt_ref[:, o:o + 1] = r


def _tc_mid(p_ref, rt_ref, bcat_ref, a0_ref, c0_ref, a1_ref, c1_ref):
    ss = []
    for o in range(ORDER):
        r = rt_ref[:, o:o + 1]
        t = (p_ref[0, o] + p_ref[1, o]) * r + bcat_ref[:, o * D:(o + 1) * D]
        ss.append(_elu(t))
    cat = jnp.concatenate(ss, axis=1)
    u = _elu(jnp.dot(cat, a0_ref[...], preferred_element_type=jnp.float32)
             + c0_ref[...])
    return jnp.dot(u, a1_ref[...], preferred_element_type=jnp.float32) + c1_ref[...]


def _tc2_body(p_ref, rt_ref, bcat_ref, a0_ref, c0_ref, a1_ref, c1_ref,
              wcat_ref, h0_ref, h1_ref, h2_ref):
    st = _tc_mid(p_ref, rt_ref, bcat_ref, a0_ref, c0_ref, a1_ref, c1_ref)
    h = jnp.dot(st, wcat_ref[...], preferred_element_type=jnp.float32)
    hs = (h0_ref, h1_ref, h2_ref)
    for o in range(ORDER):
        hs[o][...] = h[:, o * D:(o + 1) * D] * rt_ref[:, o:o + 1]


def _tc3_body(p_ref, rt_ref, bcat_ref, a0_ref, c0_ref, a1_ref, c1_ref,
              out_ref):
    st = _tc_mid(p_ref, rt_ref, bcat_ref, a0_ref, c0_ref, a1_ref, c1_ref)
    m = jnp.max(st, axis=1, keepdims=True)
    e = st - m
    lse = jnp.log(jnp.sum(jnp.exp(e), axis=1, keepdims=True))
    out_ref[...] = e - lse


def _row_spec(cols):
    return pl.BlockSpec((BN, cols), lambda i: (i, 0))


def _full_spec(shape):
    nd = len(shape)
    return pl.BlockSpec(shape, lambda i, nd=nd: (0,) * nd)


_nd_f32 = jax.ShapeDtypeStruct((N, D), jnp.float32)

_tc1 = pl.pallas_call(
    _tc1_body,
    grid=(GRID,),
    in_specs=[
        _row_spec(D),
        pl.BlockSpec((NC, ORDER, BN, D), lambda i: (0, 0, i, 0)),
        _full_spec((D, ORDER * D)),
    ],
    out_specs=[_row_spec(D)] * 3 + [_row_spec(ORDER)],
    out_shape=[_nd_f32] * 3 + [jax.ShapeDtypeStruct((N, ORDER), jnp.float32)],
)

_mid_specs = [
    pl.BlockSpec((NC, ORDER, BN, D), lambda i: (0, 0, i, 0)),
    _row_spec(ORDER),
    _full_spec((1, ORDER * D)),
    _full_spec((ORDER * D, D)),
    _full_spec((1, D)),
    _full_spec((D, D)),
    _full_spec((1, D)),
]

_tc2 = pl.pallas_call(
    _tc2_body,
    grid=(GRID,),
    in_specs=_mid_specs + [_full_spec((D, ORDER * D))],
    out_specs=[_row_spec(D)] * 3,
    out_shape=[_nd_f32] * 3,
)

_tc3 = pl.pallas_call(
    _tc3_body,
    grid=(GRID,),
    in_specs=_mid_specs,
    out_specs=_row_spec(D),
    out_shape=_nd_f32,
)


def kernel(node_feature, adj0, adj1, adj2,
           W00, b00, W01, b01, W02, b02, A0_0, c0_0, A1_0, c1_0,
           W10, b10, W11, b11, W12, b12, A0_1, c0_1, A1_1, c1_1):
    srcs = [adj0[0], adj1[0], adj2[0]]
    dsts = [adj0[1], adj1[1], adj2[1]]
    onesd = jnp.ones((CHUNK, D), jnp.float32)
    zerosd = jnp.zeros((ZROWS, D), jnp.float32)

    degp = _deg_kernel()(dsts[0], dsts[1], dsts[2], onesd, zerosd)

    wcat0 = jnp.concatenate([W00, W01, W02], axis=1)
    wcat1 = jnp.concatenate([W10, W11, W12], axis=1)
    bcat0 = jnp.concatenate([b00, b01, b02]).reshape(1, ORDER * D)
    bcat1 = jnp.concatenate([b10, b11, b12]).reshape(1, ORDER * D)

    h0, h1, h2, rt = _tc1(node_feature, degp, wcat0)
    p = _spmm_kernel()(h0, h1, h2, srcs[0], dsts[0], srcs[1], dsts[1],
                     srcs[2], dsts[2], zerosd)
    h0, h1, h2 = _tc2(p, rt, bcat0, A0_0, c0_0.reshape(1, D), A1_0,
                      c1_0.reshape(1, D), wcat1)
    p = _spmm_kernel()(h0, h1, h2, srcs[0], dsts[0], srcs[1], dsts[1],
                     srcs[2], dsts[2], zerosd)
    return _tc3(p, rt, bcat1, A0_1, c0_1.reshape(1, D), A1_1,
                c1_1.reshape(1, D))
